# baseline (speedup 1.0000x reference)
import functools

import jax
import jax.numpy as jnp
from jax import lax
from jax.experimental import pallas as pl
from jax.experimental.pallas import tpu as pltpu
from jax.experimental.pallas import tpu_sc as plsc

NUM_CORES = 2
NUM_SUBCORES = 16
NUM_WORKERS = NUM_CORES * NUM_SUBCORES
LANES = 16

BATCH = 16384
EMBED_K = 16
B_PER_W = BATCH // NUM_WORKERS
CHUNK = 128
CHUNKS = B_PER_W // CHUNK
GROUPS = B_PER_W // LANES


def _sc_body(uidx_hbm, iidx_hbm, w_hbm, h_hbm, out_hbm,
             uidx_v, iidx_v, u_v, v_v, o_v, sem_u, sem_v):
    wid = lax.axis_index("s") * NUM_CORES + lax.axis_index("c")
    row0 = wid * CHUNKS

    pltpu.sync_copy(uidx_hbm.at[pl.ds(row0, CHUNKS)], uidx_v)
    pltpu.sync_copy(iidx_hbm.at[pl.ds(row0, CHUNKS)], iidx_v)

    copies = []
    for i in range(CHUNKS):
        copies.append(pltpu.async_copy(
            w_hbm.at[uidx_v.at[i]], u_v.at[pl.ds(i * CHUNK, CHUNK)], sem_u))
        copies.append(pltpu.async_copy(
            h_hbm.at[iidx_v.at[i]], v_v.at[pl.ds(i * CHUNK, CHUNK)], sem_v))
    for cp in copies:
        cp.wait()

    lane = lax.iota(jnp.int32, LANES)

    def group_body(g, carry):
        rows = g * LANES + lane
        acc = jnp.zeros((LANES,), jnp.float32)
        for j in range(EMBED_K):
            cols = jnp.full((LANES,), j, jnp.int32)
            uu = plsc.load_gather(u_v, [rows, cols])
            vv = plsc.load_gather(v_v, [rows, cols])
            acc = acc + uu * vv
        plsc.store_scatter(o_v, [rows], acc)
        return carry

    lax.fori_loop(0, GROUPS, group_body, 0)

    pltpu.sync_copy(o_v, out_hbm.at[pl.ds(wid * B_PER_W, B_PER_W)])


@functools.partial(
    pl.kernel,
    out_type=jax.ShapeDtypeStruct((BATCH,), jnp.float32),
    mesh=plsc.VectorSubcoreMesh(core_axis_name="c", subcore_axis_name="s"),
    compiler_params=pltpu.CompilerParams(
        needs_layout_passes=False, use_tc_tiling_on_sc=False),
    scratch_types=[
        pltpu.VMEM((CHUNKS, CHUNK), jnp.int32),
        pltpu.VMEM((CHUNKS, CHUNK), jnp.int32),
        pltpu.VMEM((B_PER_W, EMBED_K), jnp.float32),
        pltpu.VMEM((B_PER_W, EMBED_K), jnp.float32),
        pltpu.VMEM((B_PER_W,), jnp.float32),
        pltpu.SemaphoreType.DMA,
        pltpu.SemaphoreType.DMA,
    ],
)
def _mf_dot_sc(uidx_hbm, iidx_hbm, w_hbm, h_hbm, out_hbm,
               uidx_v, iidx_v, u_v, v_v, o_v, sem_u, sem_v):
    _sc_body(uidx_hbm, iidx_hbm, w_hbm, h_hbm, out_hbm,
             uidx_v, iidx_v, u_v, v_v, o_v, sem_u, sem_v)


@jax.jit
def kernel(x, W, H):
    uidx = x[:, 0].reshape(BATCH // CHUNK, CHUNK).astype(jnp.int32)
    iidx = x[:, 1].reshape(BATCH // CHUNK, CHUNK).astype(jnp.int32)
    return _mf_dot_sc(uidx, iidx, W, H)
